# initial kernel scaffold (unmeasured)
import jax
import jax.numpy as jnp
from jax import lax
from jax.experimental import pallas as pl
from jax.experimental.pallas import tpu as pltpu

N_DEV = 4
SQ = 256
SKV = 4096
HQ = 32
HL = 8
DH = 128
DM = 1024
CHUNK = 512
NCHUNK = SKV // CHUNK
SCALE = 0.08838834764831843
MESH = pl.DeviceIdType.MESH
BF16 = jnp.bfloat16
F32 = jnp.float32


def kernel(x, Wq, K_ext, V_ext, Wo):
    def body(x_ref, wq_ref, k_hbm, v_hbm, wo_ref, out_ref,
             q_own, q_all, q_heads, bias,
             k_buf, v_buf, m_all, s_all, o_all,
             m_rx, s_rx, o_rx, ctx2, part_own, part_rx,
             k_sems, v_sems, qag_send, qag_recv,
             m_send, m_recv, s_send, s_recv, o_send, o_recv,
             ar_send, ar_recv):
        my = lax.axis_index("i")

        barrier = pltpu.get_barrier_semaphore()
        for i in range(N_DEV):
            @pl.when(my != i)
            def _(i=i):
                pl.semaphore_signal(barrier, inc=1, device_id=(i,),
                                    device_id_type=MESH)
        pl.semaphore_wait(barrier, N_DEV - 1)

        def kv_copy(c):
            slot = c % 2
            kc = pltpu.make_async_copy(
                k_hbm.at[0, pl.ds(c * CHUNK, CHUNK)], k_buf.at[slot],
                k_sems.at[slot])
            vc = pltpu.make_async_copy(
                v_hbm.at[0, pl.ds(c * CHUNK, CHUNK)], v_buf.at[slot],
                v_sems.at[slot])
            return kc, vc

        kc0, vc0 = kv_copy(0)
        kc0.start()
        vc0.start()

        q_own[...] = jnp.dot(
            x_ref[0].astype(BF16), wq_ref[...].astype(BF16),
            preferred_element_type=F32).astype(BF16)
        q_all[my] = q_own[...]
        for i in range(N_DEV):
            @pl.when(my != i)
            def _(i=i):
                pltpu.make_async_remote_copy(
                    src_ref=q_own, dst_ref=q_all.at[my],
                    send_sem=qag_send.at[i], recv_sem=qag_recv.at[my],
                    device_id=(i,), device_id_type=MESH).start()
        for j in range(N_DEV):
            @pl.when(my != j)
            def _(j=j):
                pltpu.make_async_remote_copy(
                    src_ref=q_own, dst_ref=q_all.at[j],
                    send_sem=qag_send.at[j], recv_sem=qag_recv.at[j],
                    device_id=(j,), device_id_type=MESH).wait_recv()
        for i in range(N_DEV):
            @pl.when(my != i)
            def _(i=i):
                pltpu.make_async_remote_copy(
                    src_ref=q_own, dst_ref=q_all.at[i],
                    send_sem=qag_send.at[i], recv_sem=qag_recv.at[i],
                    device_id=(i,), device_id_type=MESH).wait_send()

        for s in range(N_DEV):
            for j in range(HL):
                q_heads[s * HL + j] = q_all[s, :, j * DH:(j + 1) * DH]

        rows = lax.broadcasted_iota(jnp.int32, (SQ, SKV), 0)
        cols = lax.broadcasted_iota(jnp.int32, (SQ, SKV), 1)
        qb = rows // 64
        kb = my * (SKV // 64) + cols // 64
        keep = (qb == kb) | (kb == 0) | (((qb + kb) % 3) == 0)
        bias[...] = jnp.where(keep, 0.0, -1e9).astype(F32)

        m_all[...] = jnp.full((HQ, SQ), -1e30, F32)
        s_all[...] = jnp.zeros((HQ, SQ), F32)
        o_all[...] = jnp.zeros((HQ, SQ, DH), F32)

        for c in range(NCHUNK):
            slot = c % 2
            kc, vc = kv_copy(c)
            kc.wait()
            vc.wait()
            if c + 1 < NCHUNK:
                kn, vn = kv_copy(c + 1)
                kn.start()
                vn.start()

            def head_step(h, carry, c=c, slot=slot):
                q = q_heads[h]
                k = k_buf[slot, :, h, :].astype(BF16)
                v = v_buf[slot, :, h, :].astype(BF16)
                sc = lax.dot_general(q, k, (((1,), (1,)), ((), ())),
                                     preferred_element_type=F32)
                sc = sc * SCALE + bias[:, c * CHUNK:(c + 1) * CHUNK]
                m_prev = m_all[h]
                m_new = jnp.maximum(m_prev, jnp.max(sc, axis=1))
                p = jnp.exp(sc - m_new[:, None])
                alpha = jnp.exp(m_prev - m_new)
                s_all[h] = s_all[h] * alpha + jnp.sum(p, axis=1)
                o_all[h] = o_all[h] * alpha[:, None] + lax.dot_general(
                    p.astype(BF16), v, (((1,), (0,)), ((), ())),
                    preferred_element_type=F32)
                m_all[h] = m_new
                return carry

            lax.fori_loop(0, HQ, head_step, 0)

        pairs = [(m_all, m_rx, m_send, m_recv),
                 (s_all, s_rx, s_send, s_recv),
                 (o_all, o_rx, o_send, o_recv)]
        for i in range(N_DEV):
            @pl.when(my != i)
            def _(i=i):
                for src, dst, ssem, rsem in pairs:
                    pltpu.make_async_remote_copy(
                        src_ref=src.at[pl.ds(i * HL, HL)],
                        dst_ref=dst.at[my],
                        send_sem=ssem.at[i], recv_sem=rsem.at[my],
                        device_id=(i,), device_id_type=MESH).start()
        m_rx[my] = m_all[pl.ds(my * HL, HL)]
        s_rx[my] = s_all[pl.ds(my * HL, HL)]
        o_rx[my] = o_all[pl.ds(my * HL, HL)]
        for j in range(N_DEV):
            @pl.when(my != j)
            def _(j=j):
                for src, dst, ssem, rsem in pairs:
                    pltpu.make_async_remote_copy(
                        src_ref=src.at[pl.ds(j * HL, HL)],
                        dst_ref=dst.at[j],
                        send_sem=ssem.at[j], recv_sem=rsem.at[j],
                        device_id=(j,), device_id_type=MESH).wait_recv()
        for i in range(N_DEV):
            @pl.when(my != i)
            def _(i=i):
                for src, dst, ssem, rsem in pairs:
                    pltpu.make_async_remote_copy(
                        src_ref=src.at[pl.ds(i * HL, HL)],
                        dst_ref=dst.at[i],
                        send_sem=ssem.at[i], recv_sem=rsem.at[i],
                        device_id=(i,), device_id_type=MESH).wait_send()

        for j in range(HL):
            mj = m_rx[:, j, :]
            mg = jnp.max(mj, axis=0)
            al = jnp.exp(mj - mg[None, :])
            sg = jnp.sum(s_rx[:, j, :] * al, axis=0)
            oa = (o_rx[0, j] * al[0][:, None] + o_rx[1, j] * al[1][:, None]
                  + o_rx[2, j] * al[2][:, None] + o_rx[3, j] * al[3][:, None])
            ctx2[:, j * DH:(j + 1) * DH] = (oa / sg[:, None]).astype(BF16)

        part_own[...] = lax.dot_general(
            ctx2[...], wo_ref[...].astype(BF16), (((1,), (0,)), ((), ())),
            preferred_element_type=F32)
        part_rx[my] = part_own[...]
        for i in range(N_DEV):
            @pl.when(my != i)
            def _(i=i):
                pltpu.make_async_remote_copy(
                    src_ref=part_own, dst_ref=part_rx.at[my],
                    send_sem=ar_send.at[i], recv_sem=ar_recv.at[my],
                    device_id=(i,), device_id_type=MESH).start()
        for j in range(N_DEV):
            @pl.when(my != j)
            def _(j=j):
                pltpu.make_async_remote_copy(
                    src_ref=part_own, dst_ref=part_rx.at[j],
                    send_sem=ar_send.at[j], recv_sem=ar_recv.at[j],
                    device_id=(j,), device_id_type=MESH).wait_recv()
        for i in range(N_DEV):
            @pl.when(my != i)
            def _(i=i):
                pltpu.make_async_remote_copy(
                    src_ref=part_own, dst_ref=part_rx.at[i],
                    send_sem=ar_send.at[i], recv_sem=ar_recv.at[i],
                    device_id=(i,), device_id_type=MESH).wait_send()
        out_ref[0] = part_rx[0] + part_rx[1] + part_rx[2] + part_rx[3]

    return pl.pallas_call(
        body,
        out_shape=jax.ShapeDtypeStruct((1, SQ, DM), F32),
        in_specs=[
            pl.BlockSpec(memory_space=pltpu.VMEM),
            pl.BlockSpec(memory_space=pltpu.VMEM),
            pl.BlockSpec(memory_space=pltpu.ANY),
            pl.BlockSpec(memory_space=pltpu.ANY),
            pl.BlockSpec(memory_space=pltpu.VMEM),
        ],
        out_specs=pl.BlockSpec(memory_space=pltpu.VMEM),
        scratch_shapes=[
            pltpu.VMEM((SQ, DM), BF16),
            pltpu.VMEM((N_DEV, SQ, DM), BF16),
            pltpu.VMEM((HQ, SQ, DH), BF16),
            pltpu.VMEM((SQ, SKV), F32),
            pltpu.VMEM((2, CHUNK, HQ, DH), F32),
            pltpu.VMEM((2, CHUNK, HQ, DH), F32),
            pltpu.VMEM((HQ, SQ), F32),
            pltpu.VMEM((HQ, SQ), F32),
            pltpu.VMEM((HQ, SQ, DH), F32),
            pltpu.VMEM((N_DEV, HL, SQ), F32),
            pltpu.VMEM((N_DEV, HL, SQ), F32),
            pltpu.VMEM((N_DEV, HL, SQ, DH), F32),
            pltpu.VMEM((SQ, DM), BF16),
            pltpu.VMEM((SQ, DM), F32),
            pltpu.VMEM((N_DEV, SQ, DM), F32),
            pltpu.SemaphoreType.DMA((2,)),
            pltpu.SemaphoreType.DMA((2,)),
            pltpu.SemaphoreType.DMA((N_DEV,)),
            pltpu.SemaphoreType.DMA((N_DEV,)),
            pltpu.SemaphoreType.DMA((N_DEV,)),
            pltpu.SemaphoreType.DMA((N_DEV,)),
            pltpu.SemaphoreType.DMA((N_DEV,)),
            pltpu.SemaphoreType.DMA((N_DEV,)),
            pltpu.SemaphoreType.DMA((N_DEV,)),
            pltpu.SemaphoreType.DMA((N_DEV,)),
            pltpu.SemaphoreType.DMA((N_DEV,)),
            pltpu.SemaphoreType.DMA((N_DEV,)),
        ],
        compiler_params=pltpu.CompilerParams(
            collective_id=0,
            vmem_limit_bytes=128 * 1024 * 1024,
        ),
    )(x, Wq, K_ext, V_ext, Wo)


# baseline (device time: 556607 ns/iter reference)
import jax
import jax.numpy as jnp
from jax import lax
from jax.experimental import pallas as pl
from jax.experimental.pallas import tpu as pltpu

N_DEV = 4
SQ = 256
SKV = 4096
HQ = 32
HL = 8
DH = 128
DM = 1024
CHUNK = 256
NCHUNK = SKV // CHUNK
SCALE = 0.08838834764831843
MESH = pl.DeviceIdType.MESH
BF16 = jnp.bfloat16
F32 = jnp.float32


def kernel(x, Wq, K_ext, V_ext, Wo):
    def body(x_ref, wq_ref, k_hbm, v_hbm, wo_ref, out_ref,
             q_own, q_all, q_heads, bias,
             k_buf, v_buf, m_all, s_all, o_all,
             m_rx, s_rx, o_rx, ctx2, part_own, part_rx,
             k_sems, v_sems, qag_send, qag_recv,
             m_send, m_recv, s_send, s_recv, o_send, o_recv,
             ar_send, ar_recv):
        my = lax.axis_index("i")

        barrier = pltpu.get_barrier_semaphore()
        for i in range(N_DEV):
            @pl.when(my != i)
            def _(i=i):
                pl.semaphore_signal(barrier, inc=1, device_id=(i,),
                                    device_id_type=MESH)
        pl.semaphore_wait(barrier, N_DEV - 1)

        def kv_copy(c):
            slot = c % 2
            kc = pltpu.make_async_copy(
                k_hbm.at[0, pl.ds(c * CHUNK, CHUNK)], k_buf.at[slot],
                k_sems.at[slot])
            vc = pltpu.make_async_copy(
                v_hbm.at[0, pl.ds(c * CHUNK, CHUNK)], v_buf.at[slot],
                v_sems.at[slot])
            return kc, vc

        kc0, vc0 = kv_copy(0)
        kc0.start()
        vc0.start()

        q_own[...] = jnp.dot(
            x_ref[0].astype(BF16), wq_ref[...].astype(BF16),
            preferred_element_type=F32).astype(BF16)
        q_all[my] = q_own[...]
        for i in range(N_DEV):
            @pl.when(my != i)
            def _(i=i):
                pltpu.make_async_remote_copy(
                    src_ref=q_own, dst_ref=q_all.at[my],
                    send_sem=qag_send.at[i], recv_sem=qag_recv.at[my],
                    device_id=(i,), device_id_type=MESH).start()
        for j in range(N_DEV):
            @pl.when(my != j)
            def _(j=j):
                pltpu.make_async_remote_copy(
                    src_ref=q_own, dst_ref=q_all.at[j],
                    send_sem=qag_send.at[j], recv_sem=qag_recv.at[j],
                    device_id=(j,), device_id_type=MESH).wait_recv()
        for i in range(N_DEV):
            @pl.when(my != i)
            def _(i=i):
                pltpu.make_async_remote_copy(
                    src_ref=q_own, dst_ref=q_all.at[i],
                    send_sem=qag_send.at[i], recv_sem=qag_recv.at[i],
                    device_id=(i,), device_id_type=MESH).wait_send()

        for s in range(N_DEV):
            for j in range(HL):
                q_heads[s * HL + j] = q_all[s, :, j * DH:(j + 1) * DH]

        rows = lax.broadcasted_iota(jnp.int32, (SQ, SKV), 0)
        cols = lax.broadcasted_iota(jnp.int32, (SQ, SKV), 1)
        qb = rows // 64
        kb = my * (SKV // 64) + cols // 64
        keep = (qb == kb) | (kb == 0) | (((qb + kb) % 3) == 0)
        bias[...] = jnp.where(keep, 0.0, -1e9).astype(F32)

        m_all[...] = jnp.full((HQ, SQ), -1e30, F32)
        s_all[...] = jnp.zeros((HQ, SQ), F32)
        o_all[...] = jnp.zeros((HQ, SQ, DH), F32)

        for c in range(NCHUNK):
            slot = c % 2
            kc, vc = kv_copy(c)
            kc.wait()
            vc.wait()
            if c + 1 < NCHUNK:
                kn, vn = kv_copy(c + 1)
                kn.start()
                vn.start()

            def head_step(h, carry, c=c, slot=slot):
                q = q_heads[h]
                k = k_buf[slot, :, h, :].astype(BF16)
                v = v_buf[slot, :, h, :].astype(BF16)
                sc = lax.dot_general(q, k, (((1,), (1,)), ((), ())),
                                     preferred_element_type=F32)
                sc = sc * SCALE + bias[:, c * CHUNK:(c + 1) * CHUNK]
                m_prev = m_all[h]
                m_new = jnp.maximum(m_prev, jnp.max(sc, axis=1))
                p = jnp.exp(sc - m_new[:, None])
                alpha = jnp.exp(m_prev - m_new)
                s_all[h] = s_all[h] * alpha + jnp.sum(p, axis=1)
                o_all[h] = o_all[h] * alpha[:, None] + lax.dot_general(
                    p.astype(BF16), v, (((1,), (0,)), ((), ())),
                    preferred_element_type=F32)
                m_all[h] = m_new
                return carry

            lax.fori_loop(0, HQ, head_step, 0)

        pairs = [(m_all, m_rx, m_send, m_recv),
                 (s_all, s_rx, s_send, s_recv),
                 (o_all, o_rx, o_send, o_recv)]
        for i in range(N_DEV):
            @pl.when(my != i)
            def _(i=i):
                for src, dst, ssem, rsem in pairs:
                    pltpu.make_async_remote_copy(
                        src_ref=src.at[pl.ds(i * HL, HL)],
                        dst_ref=dst.at[my],
                        send_sem=ssem.at[i], recv_sem=rsem.at[my],
                        device_id=(i,), device_id_type=MESH).start()
        m_rx[my] = m_all[pl.ds(my * HL, HL)]
        s_rx[my] = s_all[pl.ds(my * HL, HL)]
        o_rx[my] = o_all[pl.ds(my * HL, HL)]
        for j in range(N_DEV):
            @pl.when(my != j)
            def _(j=j):
                for src, dst, ssem, rsem in pairs:
                    pltpu.make_async_remote_copy(
                        src_ref=src.at[pl.ds(j * HL, HL)],
                        dst_ref=dst.at[j],
                        send_sem=ssem.at[j], recv_sem=rsem.at[j],
                        device_id=(j,), device_id_type=MESH).wait_recv()
        for i in range(N_DEV):
            @pl.when(my != i)
            def _(i=i):
                for src, dst, ssem, rsem in pairs:
                    pltpu.make_async_remote_copy(
                        src_ref=src.at[pl.ds(i * HL, HL)],
                        dst_ref=dst.at[i],
                        send_sem=ssem.at[i], recv_sem=rsem.at[i],
                        device_id=(i,), device_id_type=MESH).wait_send()

        for j in range(HL):
            mj = m_rx[:, j, :]
            mg = jnp.max(mj, axis=0)
            al = jnp.exp(mj - mg[None, :])
            sg = jnp.sum(s_rx[:, j, :] * al, axis=0)
            oa = (o_rx[0, j] * al[0][:, None] + o_rx[1, j] * al[1][:, None]
                  + o_rx[2, j] * al[2][:, None] + o_rx[3, j] * al[3][:, None])
            ctx2[:, j * DH:(j + 1) * DH] = (oa / sg[:, None]).astype(BF16)

        part_own[...] = lax.dot_general(
            ctx2[...], wo_ref[...].astype(BF16), (((1,), (0,)), ((), ())),
            preferred_element_type=F32)
        part_rx[my] = part_own[...]
        for i in range(N_DEV):
            @pl.when(my != i)
            def _(i=i):
                pltpu.make_async_remote_copy(
                    src_ref=part_own, dst_ref=part_rx.at[my],
                    send_sem=ar_send.at[i], recv_sem=ar_recv.at[my],
                    device_id=(i,), device_id_type=MESH).start()
        for j in range(N_DEV):
            @pl.when(my != j)
            def _(j=j):
                pltpu.make_async_remote_copy(
                    src_ref=part_own, dst_ref=part_rx.at[j],
                    send_sem=ar_send.at[j], recv_sem=ar_recv.at[j],
                    device_id=(j,), device_id_type=MESH).wait_recv()
        for i in range(N_DEV):
            @pl.when(my != i)
            def _(i=i):
                pltpu.make_async_remote_copy(
                    src_ref=part_own, dst_ref=part_rx.at[i],
                    send_sem=ar_send.at[i], recv_sem=ar_recv.at[i],
                    device_id=(i,), device_id_type=MESH).wait_send()
        out_ref[0] = part_rx[0] + part_rx[1] + part_rx[2] + part_rx[3]

    return pl.pallas_call(
        body,
        out_shape=jax.ShapeDtypeStruct((1, SQ, DM), F32),
        in_specs=[
            pl.BlockSpec(memory_space=pltpu.VMEM),
            pl.BlockSpec(memory_space=pltpu.VMEM),
            pl.BlockSpec(memory_space=pl.ANY),
            pl.BlockSpec(memory_space=pl.ANY),
            pl.BlockSpec(memory_space=pltpu.VMEM),
        ],
        out_specs=pl.BlockSpec(memory_space=pltpu.VMEM),
        scratch_shapes=[
            pltpu.VMEM((SQ, DM), BF16),
            pltpu.VMEM((N_DEV, SQ, DM), BF16),
            pltpu.VMEM((HQ, SQ, DH), BF16),
            pltpu.VMEM((SQ, SKV), F32),
            pltpu.VMEM((2, CHUNK, HQ, DH), F32),
            pltpu.VMEM((2, CHUNK, HQ, DH), F32),
            pltpu.VMEM((HQ, SQ), F32),
            pltpu.VMEM((HQ, SQ), F32),
            pltpu.VMEM((HQ, SQ, DH), F32),
            pltpu.VMEM((N_DEV, HL, SQ), F32),
            pltpu.VMEM((N_DEV, HL, SQ), F32),
            pltpu.VMEM((N_DEV, HL, SQ, DH), F32),
            pltpu.VMEM((SQ, DM), BF16),
            pltpu.VMEM((SQ, DM), F32),
            pltpu.VMEM((N_DEV, SQ, DM), F32),
            pltpu.SemaphoreType.DMA((2,)),
            pltpu.SemaphoreType.DMA((2,)),
            pltpu.SemaphoreType.DMA((N_DEV,)),
            pltpu.SemaphoreType.DMA((N_DEV,)),
            pltpu.SemaphoreType.DMA((N_DEV,)),
            pltpu.SemaphoreType.DMA((N_DEV,)),
            pltpu.SemaphoreType.DMA((N_DEV,)),
            pltpu.SemaphoreType.DMA((N_DEV,)),
            pltpu.SemaphoreType.DMA((N_DEV,)),
            pltpu.SemaphoreType.DMA((N_DEV,)),
            pltpu.SemaphoreType.DMA((N_DEV,)),
            pltpu.SemaphoreType.DMA((N_DEV,)),
        ],
        compiler_params=pltpu.CompilerParams(
            collective_id=0,
            vmem_limit_bytes=128 * 1024 * 1024,
        ),
    )(x, Wq, K_ext, V_ext, Wo)


# device time: 159716 ns/iter; 3.4850x vs baseline; 3.4850x over previous
import jax
import jax.numpy as jnp
from jax import lax
from jax.experimental import pallas as pl
from jax.experimental.pallas import tpu as pltpu

N_DEV = 4
SQ = 256
SKV = 4096
HQ = 32
HL = 8
DH = 128
DM = 1024
CHUNK = 256
NCHUNK = SKV // CHUNK
SCALE = 0.08838834764831843
MESH = pl.DeviceIdType.MESH
BF16 = jnp.bfloat16
F32 = jnp.float32


def kernel(x, Wq, K_ext, V_ext, Wo):
    def body(x_ref, wq_ref, k_hbm, v_hbm, wo_ref, out_ref,
             q_own, q_all, q_heads, bias,
             k_buf, v_buf, m_all, s_all, o_all,
             m_rx, s_rx, o_rx, ctx2, part_own, part_rx,
             k_sems, v_sems, qag_send, qag_recv,
             m_send, m_recv, s_send, s_recv, o_send, o_recv,
             ar_send, ar_recv):
        my = lax.axis_index("i")

        barrier = pltpu.get_barrier_semaphore()
        for i in range(N_DEV):
            @pl.when(my != i)
            def _(i=i):
                pl.semaphore_signal(barrier, inc=1, device_id=(i,),
                                    device_id_type=MESH)
        pl.semaphore_wait(barrier, N_DEV - 1)

        def kv_copy(h):
            slot = lax.rem(h, 2)
            kc = pltpu.make_async_copy(
                k_hbm.at[0, :, h, :], k_buf.at[slot], k_sems.at[slot])
            vc = pltpu.make_async_copy(
                v_hbm.at[0, :, h, :], v_buf.at[slot], v_sems.at[slot])
            return kc, vc

        kc0, vc0 = kv_copy(0)
        kc0.start()
        vc0.start()

        q_own[...] = jnp.dot(
            x_ref[0].astype(BF16), wq_ref[...].astype(BF16),
            preferred_element_type=F32).astype(BF16)
        q_all[my] = q_own[...]
        for i in range(N_DEV):
            @pl.when(my != i)
            def _(i=i):
                pltpu.make_async_remote_copy(
                    src_ref=q_own, dst_ref=q_all.at[my],
                    send_sem=qag_send.at[i], recv_sem=qag_recv.at[my],
                    device_id=(i,), device_id_type=MESH).start()
        for j in range(N_DEV):
            @pl.when(my != j)
            def _(j=j):
                pltpu.make_async_remote_copy(
                    src_ref=q_own, dst_ref=q_all.at[j],
                    send_sem=qag_send.at[j], recv_sem=qag_recv.at[j],
                    device_id=(j,), device_id_type=MESH).wait_recv()
        for i in range(N_DEV):
            @pl.when(my != i)
            def _(i=i):
                pltpu.make_async_remote_copy(
                    src_ref=q_own, dst_ref=q_all.at[i],
                    send_sem=qag_send.at[i], recv_sem=qag_recv.at[i],
                    device_id=(i,), device_id_type=MESH).wait_send()

        for s in range(N_DEV):
            for j in range(HL):
                q_heads[s * HL + j] = q_all[s, :, j * DH:(j + 1) * DH]

        rows = lax.broadcasted_iota(jnp.int32, (SQ, SKV), 0)
        cols = lax.broadcasted_iota(jnp.int32, (SQ, SKV), 1)
        qb = rows // 64
        kb = my * (SKV // 64) + cols // 64
        keep = (qb == kb) | (kb == 0) | (((qb + kb) % 3) == 0)
        bias[...] = jnp.where(keep, 0.0, -1e9).astype(F32)

        def head_step(h, carry):
            slot = lax.rem(h, 2)
            kc, vc = kv_copy(h)
            kc.wait()
            vc.wait()

            @pl.when(h < HQ - 1)
            def _():
                kn, vn = kv_copy(h + 1)
                kn.start()
                vn.start()

            q = q_heads[h]
            k = k_buf[slot].astype(BF16)
            v = v_buf[slot].astype(BF16)
            sc = lax.dot_general(q, k, (((1,), (1,)), ((), ())),
                                 preferred_element_type=F32)
            sc = sc * SCALE + bias[...]
            m = jnp.max(sc, axis=1)
            p = jnp.exp(sc - m[:, None])
            s_all[h] = jnp.sum(p, axis=1)
            o_all[h] = lax.dot_general(
                p.astype(BF16), v, (((1,), (0,)), ((), ())),
                preferred_element_type=F32)
            m_all[h] = m
            return carry

        lax.fori_loop(0, HQ, head_step, 0)

        pairs = [(m_all, m_rx, m_send, m_recv),
                 (s_all, s_rx, s_send, s_recv),
                 (o_all, o_rx, o_send, o_recv)]
        for i in range(N_DEV):
            @pl.when(my != i)
            def _(i=i):
                for src, dst, ssem, rsem in pairs:
                    pltpu.make_async_remote_copy(
                        src_ref=src.at[pl.ds(i * HL, HL)],
                        dst_ref=dst.at[my],
                        send_sem=ssem.at[i], recv_sem=rsem.at[my],
                        device_id=(i,), device_id_type=MESH).start()
        m_rx[my] = m_all[pl.ds(my * HL, HL)]
        s_rx[my] = s_all[pl.ds(my * HL, HL)]
        o_rx[my] = o_all[pl.ds(my * HL, HL)]
        for j in range(N_DEV):
            @pl.when(my != j)
            def _(j=j):
                for src, dst, ssem, rsem in pairs:
                    pltpu.make_async_remote_copy(
                        src_ref=src.at[pl.ds(j * HL, HL)],
                        dst_ref=dst.at[j],
                        send_sem=ssem.at[j], recv_sem=rsem.at[j],
                        device_id=(j,), device_id_type=MESH).wait_recv()
        for i in range(N_DEV):
            @pl.when(my != i)
            def _(i=i):
                for src, dst, ssem, rsem in pairs:
                    pltpu.make_async_remote_copy(
                        src_ref=src.at[pl.ds(i * HL, HL)],
                        dst_ref=dst.at[i],
                        send_sem=ssem.at[i], recv_sem=rsem.at[i],
                        device_id=(i,), device_id_type=MESH).wait_send()

        for j in range(HL):
            mj = m_rx[:, j, :]
            mg = jnp.max(mj, axis=0)
            al = jnp.exp(mj - mg[None, :])
            sg = jnp.sum(s_rx[:, j, :] * al, axis=0)
            oa = (o_rx[0, j] * al[0][:, None] + o_rx[1, j] * al[1][:, None]
                  + o_rx[2, j] * al[2][:, None] + o_rx[3, j] * al[3][:, None])
            ctx2[:, j * DH:(j + 1) * DH] = (oa / sg[:, None]).astype(BF16)

        part_own[...] = lax.dot_general(
            ctx2[...], wo_ref[...].astype(BF16), (((1,), (0,)), ((), ())),
            preferred_element_type=F32)
        part_rx[my] = part_own[...]
        for i in range(N_DEV):
            @pl.when(my != i)
            def _(i=i):
                pltpu.make_async_remote_copy(
                    src_ref=part_own, dst_ref=part_rx.at[my],
                    send_sem=ar_send.at[i], recv_sem=ar_recv.at[my],
                    device_id=(i,), device_id_type=MESH).start()
        for j in range(N_DEV):
            @pl.when(my != j)
            def _(j=j):
                pltpu.make_async_remote_copy(
                    src_ref=part_own, dst_ref=part_rx.at[j],
                    send_sem=ar_send.at[j], recv_sem=ar_recv.at[j],
                    device_id=(j,), device_id_type=MESH).wait_recv()
        for i in range(N_DEV):
            @pl.when(my != i)
            def _(i=i):
                pltpu.make_async_remote_copy(
                    src_ref=part_own, dst_ref=part_rx.at[i],
                    send_sem=ar_send.at[i], recv_sem=ar_recv.at[i],
                    device_id=(i,), device_id_type=MESH).wait_send()
        out_ref[0] = part_rx[0] + part_rx[1] + part_rx[2] + part_rx[3]

    return pl.pallas_call(
        body,
        out_shape=jax.ShapeDtypeStruct((1, SQ, DM), F32),
        in_specs=[
            pl.BlockSpec(memory_space=pltpu.VMEM),
            pl.BlockSpec(memory_space=pltpu.VMEM),
            pl.BlockSpec(memory_space=pl.ANY),
            pl.BlockSpec(memory_space=pl.ANY),
            pl.BlockSpec(memory_space=pltpu.VMEM),
        ],
        out_specs=pl.BlockSpec(memory_space=pltpu.VMEM),
        scratch_shapes=[
            pltpu.VMEM((SQ, DM), BF16),
            pltpu.VMEM((N_DEV, SQ, DM), BF16),
            pltpu.VMEM((HQ, SQ, DH), BF16),
            pltpu.VMEM((SQ, SKV), F32),
            pltpu.VMEM((2, SKV, DH), F32),
            pltpu.VMEM((2, SKV, DH), F32),
            pltpu.VMEM((HQ, SQ), F32),
            pltpu.VMEM((HQ, SQ), F32),
            pltpu.VMEM((HQ, SQ, DH), F32),
            pltpu.VMEM((N_DEV, HL, SQ), F32),
            pltpu.VMEM((N_DEV, HL, SQ), F32),
            pltpu.VMEM((N_DEV, HL, SQ, DH), F32),
            pltpu.VMEM((SQ, DM), BF16),
            pltpu.VMEM((SQ, DM), F32),
            pltpu.VMEM((N_DEV, SQ, DM), F32),
            pltpu.SemaphoreType.DMA((2,)),
            pltpu.SemaphoreType.DMA((2,)),
            pltpu.SemaphoreType.DMA((N_DEV,)),
            pltpu.SemaphoreType.DMA((N_DEV,)),
            pltpu.SemaphoreType.DMA((N_DEV,)),
            pltpu.SemaphoreType.DMA((N_DEV,)),
            pltpu.SemaphoreType.DMA((N_DEV,)),
            pltpu.SemaphoreType.DMA((N_DEV,)),
            pltpu.SemaphoreType.DMA((N_DEV,)),
            pltpu.SemaphoreType.DMA((N_DEV,)),
            pltpu.SemaphoreType.DMA((N_DEV,)),
            pltpu.SemaphoreType.DMA((N_DEV,)),
        ],
        compiler_params=pltpu.CompilerParams(
            collective_id=0,
            vmem_limit_bytes=128 * 1024 * 1024,
        ),
    )(x, Wq, K_ext, V_ext, Wo)


# device time: 111847 ns/iter; 4.9765x vs baseline; 1.4280x over previous
import jax
import jax.numpy as jnp
from jax import lax
from jax.experimental import pallas as pl
from jax.experimental.pallas import tpu as pltpu

N_DEV = 4
SQ = 256
SKV = 4096
HQ = 32
HL = 8
DH = 128
DM = 1024
SCALE = 0.08838834764831843
MESH = pl.DeviceIdType.MESH
BF16 = jnp.bfloat16
F32 = jnp.float32


def kernel(x, Wq, K_ext, V_ext, Wo):
    def body(x_ref, wq_ref, k_hbm, v_hbm, wo_ref, out_ref,
             q_own, q_all, q_heads, bias,
             k_buf, v_buf, s_all, o_all,
             s_rx, o_rx, ctx2, part_own, part_rx,
             k_sems, v_sems, qag_send, qag_recv,
             s_send, s_recv, o_send, o_recv,
             ar_send, ar_recv):
        my = lax.axis_index("i")

        barrier = pltpu.get_barrier_semaphore()
        for i in range(N_DEV):
            @pl.when(my != i)
            def _(i=i):
                pl.semaphore_signal(barrier, inc=1, device_id=(i,),
                                    device_id_type=MESH)
        pl.semaphore_wait(barrier, N_DEV - 1)

        def kv_copy(h):
            slot = lax.rem(h, 2)
            kc = pltpu.make_async_copy(
                k_hbm.at[0, :, h, :], k_buf.at[slot], k_sems.at[slot])
            vc = pltpu.make_async_copy(
                v_hbm.at[0, :, h, :], v_buf.at[slot], v_sems.at[slot])
            return kc, vc

        kc0, vc0 = kv_copy(my * HL)
        kc0.start()
        vc0.start()

        q_own[...] = (jnp.dot(
            x_ref[0].astype(BF16), wq_ref[...].astype(BF16),
            preferred_element_type=F32) * SCALE).astype(BF16)
        for i in range(N_DEV):
            @pl.when(my != i)
            def _(i=i):
                pltpu.make_async_remote_copy(
                    src_ref=q_own, dst_ref=q_all.at[my],
                    send_sem=qag_send.at[i], recv_sem=qag_recv.at[my],
                    device_id=(i,), device_id_type=MESH).start()

        rows = lax.broadcasted_iota(jnp.int32, (SQ, SKV), 0)
        cols = lax.broadcasted_iota(jnp.int32, (SQ, SKV), 1)
        qb = rows // 64
        kb = my * (SKV // 64) + cols // 64
        keep = (qb == kb) | (kb == 0) | (((qb + kb) % 3) == 0)
        bias[...] = jnp.where(keep, 0.0, -1e9).astype(BF16)

        pairs = [(s_all, s_rx, s_send, s_recv),
                 (o_all, o_rx, o_send, o_recv)]
        for t in range(N_DEV):
            g = lax.rem(my + t, N_DEV)
            if t == 0:
                for j in range(HL):
                    q_heads[my * HL + j] = q_own[:, j * DH:(j + 1) * DH]
            else:
                pltpu.make_async_remote_copy(
                    src_ref=q_own, dst_ref=q_all.at[g],
                    send_sem=qag_send.at[g], recv_sem=qag_recv.at[g],
                    device_id=(g,), device_id_type=MESH).wait_recv()
                for j in range(HL):
                    q_heads[g * HL + j] = q_all[g, :, j * DH:(j + 1) * DH]

            def grp_step(jj, carry, t=t):
                g_ = lax.rem(my + t, N_DEV)
                h = g_ * HL + jj
                slot = lax.rem(jj, 2)
                kc, vc = kv_copy(h)
                kc.wait()
                vc.wait()

                if t < N_DEV - 1:
                    nxt = jnp.where(jj < HL - 1, h + 1,
                                    lax.rem(my + t + 1, N_DEV) * HL)
                    kn, vn = kv_copy(nxt)
                    kn.start()
                    vn.start()
                else:
                    @pl.when(jj < HL - 1)
                    def _():
                        kn, vn = kv_copy(h + 1)
                        kn.start()
                        vn.start()

                q = q_heads[h]
                k = k_buf[slot].astype(BF16)
                v = v_buf[slot].astype(BF16)
                sc = lax.dot_general(q, k, (((1,), (1,)), ((), ())),
                                     preferred_element_type=F32)
                p = jnp.exp(sc + bias[...].astype(F32)).astype(BF16)
                s_all[h] = jnp.sum(p, axis=1, dtype=F32)
                o_all[h] = lax.dot_general(
                    p, v, (((1,), (0,)), ((), ())),
                    preferred_element_type=F32).astype(BF16)
                return carry

            lax.fori_loop(0, HL, grp_step, 0)

            if t == 0:
                s_rx[my] = s_all[pl.ds(my * HL, HL)]
                o_rx[my] = o_all[pl.ds(my * HL, HL)]
            else:
                for src, dst, ssem, rsem in pairs:
                    pltpu.make_async_remote_copy(
                        src_ref=src.at[pl.ds(g * HL, HL)],
                        dst_ref=dst.at[my],
                        send_sem=ssem.at[g], recv_sem=rsem.at[my],
                        device_id=(g,), device_id_type=MESH).start()

        for j in range(N_DEV):
            @pl.when(my != j)
            def _(j=j):
                for src, dst, ssem, rsem in pairs:
                    pltpu.make_async_remote_copy(
                        src_ref=src.at[pl.ds(j * HL, HL)],
                        dst_ref=dst.at[j],
                        send_sem=ssem.at[j], recv_sem=rsem.at[j],
                        device_id=(j,), device_id_type=MESH).wait_recv()
        for i in range(N_DEV):
            @pl.when(my != i)
            def _(i=i):
                for src, dst, ssem, rsem in pairs:
                    pltpu.make_async_remote_copy(
                        src_ref=src.at[pl.ds(i * HL, HL)],
                        dst_ref=dst.at[i],
                        send_sem=ssem.at[i], recv_sem=rsem.at[i],
                        device_id=(i,), device_id_type=MESH).wait_send()

        for j in range(HL):
            sg = (s_rx[0, j] + s_rx[1, j] + s_rx[2, j] + s_rx[3, j])
            oa = (o_rx[0, j].astype(F32) + o_rx[1, j].astype(F32)
                  + o_rx[2, j].astype(F32) + o_rx[3, j].astype(F32))
            ctx2[:, j * DH:(j + 1) * DH] = (oa / sg[:, None]).astype(BF16)

        part_own[...] = lax.dot_general(
            ctx2[...], wo_ref[...].astype(BF16), (((1,), (0,)), ((), ())),
            preferred_element_type=F32).astype(BF16)
        part_rx[my] = part_own[...]
        for i in range(N_DEV):
            @pl.when(my != i)
            def _(i=i):
                pltpu.make_async_remote_copy(
                    src_ref=part_own, dst_ref=part_rx.at[my],
                    send_sem=ar_send.at[i], recv_sem=ar_recv.at[my],
                    device_id=(i,), device_id_type=MESH).start()
        for j in range(N_DEV):
            @pl.when(my != j)
            def _(j=j):
                pltpu.make_async_remote_copy(
                    src_ref=part_own, dst_ref=part_rx.at[j],
                    send_sem=ar_send.at[j], recv_sem=ar_recv.at[j],
                    device_id=(j,), device_id_type=MESH).wait_recv()
        for i in range(N_DEV):
            @pl.when(my != i)
            def _(i=i):
                pltpu.make_async_remote_copy(
                    src_ref=part_own, dst_ref=part_rx.at[i],
                    send_sem=ar_send.at[i], recv_sem=ar_recv.at[i],
                    device_id=(i,), device_id_type=MESH).wait_send()
        for i in range(N_DEV):
            @pl.when(my != i)
            def _(i=i):
                pltpu.make_async_remote_copy(
                    src_ref=q_own, dst_ref=q_all.at[i],
                    send_sem=qag_send.at[i], recv_sem=qag_recv.at[i],
                    device_id=(i,), device_id_type=MESH).wait_send()
        out_ref[0] = (part_rx[0].astype(F32) + part_rx[1].astype(F32)
                      + part_rx[2].astype(F32) + part_rx[3].astype(F32))

    return pl.pallas_call(
        body,
        out_shape=jax.ShapeDtypeStruct((1, SQ, DM), F32),
        in_specs=[
            pl.BlockSpec(memory_space=pltpu.VMEM),
            pl.BlockSpec(memory_space=pltpu.VMEM),
            pl.BlockSpec(memory_space=pl.ANY),
            pl.BlockSpec(memory_space=pl.ANY),
            pl.BlockSpec(memory_space=pltpu.VMEM),
        ],
        out_specs=pl.BlockSpec(memory_space=pltpu.VMEM),
        scratch_shapes=[
            pltpu.VMEM((SQ, DM), BF16),
            pltpu.VMEM((N_DEV, SQ, DM), BF16),
            pltpu.VMEM((HQ, SQ, DH), BF16),
            pltpu.VMEM((SQ, SKV), BF16),
            pltpu.VMEM((2, SKV, DH), F32),
            pltpu.VMEM((2, SKV, DH), F32),
            pltpu.VMEM((HQ, SQ), F32),
            pltpu.VMEM((HQ, SQ, DH), BF16),
            pltpu.VMEM((N_DEV, HL, SQ), F32),
            pltpu.VMEM((N_DEV, HL, SQ, DH), BF16),
            pltpu.VMEM((SQ, DM), BF16),
            pltpu.VMEM((SQ, DM), BF16),
            pltpu.VMEM((N_DEV, SQ, DM), BF16),
            pltpu.SemaphoreType.DMA((2,)),
            pltpu.SemaphoreType.DMA((2,)),
            pltpu.SemaphoreType.DMA((N_DEV,)),
            pltpu.SemaphoreType.DMA((N_DEV,)),
            pltpu.SemaphoreType.DMA((N_DEV,)),
            pltpu.SemaphoreType.DMA((N_DEV,)),
            pltpu.SemaphoreType.DMA((N_DEV,)),
            pltpu.SemaphoreType.DMA((N_DEV,)),
            pltpu.SemaphoreType.DMA((N_DEV,)),
            pltpu.SemaphoreType.DMA((N_DEV,)),
        ],
        compiler_params=pltpu.CompilerParams(
            collective_id=0,
            vmem_limit_bytes=128 * 1024 * 1024,
        ),
    )(x, Wq, K_ext, V_ext, Wo)
